# baseline (speedup 1.0000x reference)
import functools

import numpy as np
import jax
import jax.numpy as jnp
from jax import lax
from jax.experimental import pallas as pl
from jax.experimental.pallas import tpu as pltpu
from jax.experimental.pallas import tpu_sc as plsc

VOCAB = 28996
EMB = 512
SEQ = 8192
BASE_FREQ = 1e-05

NUM_CORES = 2
NUM_SUBCORES = 16
NW = NUM_CORES * NUM_SUBCORES
ROWS_PER_W = SEQ // NW
CHUNK = 32
NCHUNK = ROWS_PER_W // CHUNK
NBUF = 4
LANES = 16
NBLK = EMB // LANES
NSEED = 8
AUX = (NSEED + 1) * LANES


def _aux_host() -> np.ndarray:
    m = np.float64(BASE_FREQ) ** np.linspace(0.0, 2.0, SEQ, dtype=np.float64)
    c = np.arange(NSEED * LANES, dtype=np.float64)
    seeds = np.sin(c[None, :] * m[:, None])
    c2 = np.broadcast_to(
        2.0 * np.cos(NSEED // 2 * LANES * m)[:, None], (SEQ, LANES))
    return np.concatenate([seeds, c2], axis=1).astype(np.float32)


_AUX = _aux_host()


@functools.partial(
    pl.kernel,
    mesh=plsc.VectorSubcoreMesh(core_axis_name="c", subcore_axis_name="s"),
    out_type=jax.ShapeDtypeStruct((SEQ, EMB), jnp.float32),
    scratch_types=[
        pltpu.VMEM((ROWS_PER_W,), jnp.int32),
        pltpu.VMEM((NBUF, CHUNK, EMB), jnp.float32),
        pltpu.VMEM((NBUF, CHUNK, AUX), jnp.float32),
    ] + [pltpu.SemaphoreType.DMA] * (3 * NBUF),
)
def _encode(table_hbm, x_hbm, aux_hbm, out_hbm, idx_v, rows_v, aux_v, *sems):
    sem_g = sems[0:NBUF]
    sem_a = sems[NBUF:2 * NBUF]
    sem_out = sems[2 * NBUF:3 * NBUF]
    wid = lax.axis_index("s") * NUM_CORES + lax.axis_index("c")
    base = wid * ROWS_PER_W
    pltpu.sync_copy(x_hbm.at[pl.ds(base, ROWS_PER_W)], idx_v)

    def fetch_start(k):
        b = k % NBUF
        g = pltpu.async_copy(
            table_hbm.at[idx_v.at[pl.ds(k * CHUNK, CHUNK)]], rows_v.at[b],
            sem_g[b])
        a = pltpu.async_copy(
            aux_hbm.at[pl.ds(base + k * CHUNK, CHUNK)], aux_v.at[b], sem_a[b])
        return g, a

    pending = {k: fetch_start(k) for k in range(min(NBUF - 1, NCHUNK))}
    pending_out = {}
    for k in range(NCHUNK):
        b = k % NBUF
        if k + NBUF - 1 < NCHUNK:
            if k >= 1:
                pending_out.pop(k - 1).wait()
            pending[k + NBUF - 1] = fetch_start(k + NBUF - 1)
        g, a = pending.pop(k)
        g.wait()
        a.wait()

        @plsc.parallel_loop(0, CHUNK, unroll=1)
        def _row(r):
            c2 = aux_v[b, r, pl.ds(NSEED * LANES, LANES)]
            ring = []
            for j in range(NSEED):
                vj = aux_v[b, r, pl.ds(j * LANES, LANES)]
                s = pl.ds(j * LANES, LANES)
                rows_v[b, r, s] = rows_v[b, r, s] + vj
                ring.append(vj)
            half = NSEED // 2
            for j in range(NSEED, NBLK):
                vn = c2 * ring[-half] - ring[-NSEED]
                s = pl.ds(j * LANES, LANES)
                rows_v[b, r, s] = rows_v[b, r, s] + vn
                ring.append(vn)
                ring.pop(0)

        pending_out[k] = pltpu.async_copy(
            rows_v.at[b], out_hbm.at[pl.ds(base + k * CHUNK, CHUNK)], sem_out[b])
    for k in sorted(pending_out):
        pending_out[k].wait()


def kernel(x, table):
    aux = jnp.asarray(_AUX)
    return _encode(table, x, aux)


# speedup vs baseline: 1.0766x; 1.0766x over previous
import functools

import numpy as np
import jax
import jax.numpy as jnp
from jax import lax
from jax.experimental import pallas as pl
from jax.experimental.pallas import tpu as pltpu
from jax.experimental.pallas import tpu_sc as plsc

VOCAB = 28996
EMB = 512
SEQ = 8192
BASE_FREQ = 1e-05

NUM_CORES = 2
NUM_SUBCORES = 16
NW = NUM_CORES * NUM_SUBCORES
ROWS_PER_W = SEQ // NW
CHUNK = 32
NCHUNK = ROWS_PER_W // CHUNK
NBUF = 4
LANES = 16
NBLK = EMB // LANES
NSEED = 2
AUX = (NSEED + 1) * LANES


def _aux_host() -> np.ndarray:
    m = np.float64(BASE_FREQ) ** np.linspace(0.0, 2.0, SEQ, dtype=np.float64)
    c = np.arange(NSEED * LANES, dtype=np.float64)
    seeds = np.sin(c[None, :] * m[:, None])
    c2 = np.broadcast_to(
        2.0 * np.cos(NSEED // 2 * LANES * m)[:, None], (SEQ, LANES))
    return np.concatenate([seeds, c2], axis=1).astype(np.float32)


_AUX = _aux_host()


@functools.partial(
    pl.kernel,
    mesh=plsc.VectorSubcoreMesh(core_axis_name="c", subcore_axis_name="s"),
    out_type=jax.ShapeDtypeStruct((SEQ, EMB), jnp.float32),
    scratch_types=[
        pltpu.VMEM((ROWS_PER_W,), jnp.int32),
        pltpu.VMEM((NBUF, CHUNK, EMB), jnp.float32),
        pltpu.VMEM((NBUF, CHUNK, AUX), jnp.float32),
    ] + [pltpu.SemaphoreType.DMA] * (3 * NBUF),
)
def _encode(table_hbm, x_hbm, aux_hbm, out_hbm, idx_v, rows_v, aux_v, *sems):
    sem_g = sems[0:NBUF]
    sem_a = sems[NBUF:2 * NBUF]
    sem_out = sems[2 * NBUF:3 * NBUF]
    wid = lax.axis_index("s") * NUM_CORES + lax.axis_index("c")
    base = wid * ROWS_PER_W
    pltpu.sync_copy(x_hbm.at[pl.ds(base, ROWS_PER_W)], idx_v)

    def fetch_start(k):
        b = k % NBUF
        g = pltpu.async_copy(
            table_hbm.at[idx_v.at[pl.ds(k * CHUNK, CHUNK)]], rows_v.at[b],
            sem_g[b])
        a = pltpu.async_copy(
            aux_hbm.at[pl.ds(base + k * CHUNK, CHUNK)], aux_v.at[b], sem_a[b])
        return g, a

    pending = {k: fetch_start(k) for k in range(min(NBUF - 1, NCHUNK))}
    pending_out = {}
    for k in range(NCHUNK):
        b = k % NBUF
        if k + NBUF - 1 < NCHUNK:
            if k >= 1:
                pending_out.pop(k - 1).wait()
            pending[k + NBUF - 1] = fetch_start(k + NBUF - 1)
        g, a = pending.pop(k)
        g.wait()
        a.wait()

        @plsc.parallel_loop(0, CHUNK, unroll=1)
        def _row(r):
            c2 = aux_v[b, r, pl.ds(NSEED * LANES, LANES)]
            ring = []
            for j in range(NSEED):
                vj = aux_v[b, r, pl.ds(j * LANES, LANES)]
                s = pl.ds(j * LANES, LANES)
                rows_v[b, r, s] = rows_v[b, r, s] + vj
                ring.append(vj)
            half = NSEED // 2
            for j in range(NSEED, NBLK):
                vn = c2 * ring[-half] - ring[-NSEED]
                s = pl.ds(j * LANES, LANES)
                rows_v[b, r, s] = rows_v[b, r, s] + vn
                ring.append(vn)
                ring.pop(0)

        pending_out[k] = pltpu.async_copy(
            rows_v.at[b], out_hbm.at[pl.ds(base + k * CHUNK, CHUNK)], sem_out[b])
    for k in sorted(pending_out):
        pending_out[k].wait()


def kernel(x, table):
    aux = jnp.asarray(_AUX)
    return _encode(table, x, aux)


# speedup vs baseline: 1.0836x; 1.0065x over previous
import functools

import numpy as np
import jax
import jax.numpy as jnp
from jax import lax
from jax.experimental import pallas as pl
from jax.experimental.pallas import tpu as pltpu
from jax.experimental.pallas import tpu_sc as plsc

VOCAB = 28996
EMB = 512
SEQ = 8192
BASE_FREQ = 1e-05

NUM_CORES = 2
NUM_SUBCORES = 16
NW = NUM_CORES * NUM_SUBCORES
ROWS_PER_W = SEQ // NW
CHUNK = 32
NCHUNK = ROWS_PER_W // CHUNK
NBUF = 4
LANES = 16
NBLK = EMB // LANES
NSEED = 4
AUX = (NSEED + 1) * LANES


def _aux_host() -> np.ndarray:
    m = np.float64(BASE_FREQ) ** np.linspace(0.0, 2.0, SEQ, dtype=np.float64)
    c = np.arange(NSEED * LANES, dtype=np.float64)
    seeds = np.sin(c[None, :] * m[:, None])
    c2 = np.broadcast_to(
        2.0 * np.cos(NSEED // 2 * LANES * m)[:, None], (SEQ, LANES))
    return np.concatenate([seeds, c2], axis=1).astype(np.float32)


_AUX = _aux_host()


@functools.partial(
    pl.kernel,
    mesh=plsc.VectorSubcoreMesh(core_axis_name="c", subcore_axis_name="s"),
    out_type=jax.ShapeDtypeStruct((SEQ, EMB), jnp.float32),
    scratch_types=[
        pltpu.VMEM((ROWS_PER_W,), jnp.int32),
        pltpu.VMEM((NBUF, CHUNK, EMB), jnp.float32),
        pltpu.VMEM((NBUF, CHUNK, AUX), jnp.float32),
    ] + [pltpu.SemaphoreType.DMA] * (3 * NBUF),
)
def _encode(table_hbm, x_hbm, aux_hbm, out_hbm, idx_v, rows_v, aux_v, *sems):
    sem_g = sems[0:NBUF]
    sem_a = sems[NBUF:2 * NBUF]
    sem_out = sems[2 * NBUF:3 * NBUF]
    wid = lax.axis_index("s") * NUM_CORES + lax.axis_index("c")
    base = wid * ROWS_PER_W
    pltpu.sync_copy(x_hbm.at[pl.ds(base, ROWS_PER_W)], idx_v)

    def fetch_start(k):
        b = k % NBUF
        g = pltpu.async_copy(
            table_hbm.at[idx_v.at[pl.ds(k * CHUNK, CHUNK)]], rows_v.at[b],
            sem_g[b])
        a = pltpu.async_copy(
            aux_hbm.at[pl.ds(base + k * CHUNK, CHUNK)], aux_v.at[b], sem_a[b])
        return g, a

    pending = {k: fetch_start(k) for k in range(min(NBUF - 1, NCHUNK))}
    pending_out = {}
    for k in range(NCHUNK):
        b = k % NBUF
        if k + NBUF - 1 < NCHUNK:
            if k >= 1:
                pending_out.pop(k - 1).wait()
            pending[k + NBUF - 1] = fetch_start(k + NBUF - 1)
        g, a = pending.pop(k)
        g.wait()
        a.wait()

        @plsc.parallel_loop(0, CHUNK, unroll=1)
        def _row(r):
            c2 = aux_v[b, r, pl.ds(NSEED * LANES, LANES)]
            ring = []
            for j in range(NSEED):
                vj = aux_v[b, r, pl.ds(j * LANES, LANES)]
                s = pl.ds(j * LANES, LANES)
                rows_v[b, r, s] = rows_v[b, r, s] + vj
                ring.append(vj)
            half = NSEED // 2
            for j in range(NSEED, NBLK):
                vn = c2 * ring[-half] - ring[-NSEED]
                s = pl.ds(j * LANES, LANES)
                rows_v[b, r, s] = rows_v[b, r, s] + vn
                ring.append(vn)
                ring.pop(0)

        pending_out[k] = pltpu.async_copy(
            rows_v.at[b], out_hbm.at[pl.ds(base + k * CHUNK, CHUNK)], sem_out[b])
    for k in sorted(pending_out):
        pending_out[k].wait()


def kernel(x, table):
    aux = jnp.asarray(_AUX)
    return _encode(table, x, aux)


# speedup vs baseline: 1.0918x; 1.0075x over previous
import functools

import numpy as np
import jax
import jax.numpy as jnp
from jax import lax
from jax.experimental import pallas as pl
from jax.experimental.pallas import tpu as pltpu
from jax.experimental.pallas import tpu_sc as plsc

VOCAB = 28996
EMB = 512
SEQ = 8192
BASE_FREQ = 1e-05

NUM_CORES = 2
NUM_SUBCORES = 16
NW = NUM_CORES * NUM_SUBCORES
ROWS_PER_W = SEQ // NW
CHUNK = 64
NCHUNK = ROWS_PER_W // CHUNK
NBUF = 3
LANES = 16
NBLK = EMB // LANES
NSEED = 4
AUX = (NSEED + 1) * LANES


def _aux_host() -> np.ndarray:
    m = np.float64(BASE_FREQ) ** np.linspace(0.0, 2.0, SEQ, dtype=np.float64)
    c = np.arange(NSEED * LANES, dtype=np.float64)
    seeds = np.sin(c[None, :] * m[:, None])
    c2 = np.broadcast_to(
        2.0 * np.cos(NSEED // 2 * LANES * m)[:, None], (SEQ, LANES))
    return np.concatenate([seeds, c2], axis=1).astype(np.float32)


_AUX = _aux_host()


@functools.partial(
    pl.kernel,
    mesh=plsc.VectorSubcoreMesh(core_axis_name="c", subcore_axis_name="s"),
    out_type=jax.ShapeDtypeStruct((SEQ, EMB), jnp.float32),
    scratch_types=[
        pltpu.VMEM((ROWS_PER_W,), jnp.int32),
        pltpu.VMEM((NBUF, CHUNK, EMB), jnp.float32),
        pltpu.VMEM((NBUF, CHUNK, AUX), jnp.float32),
    ] + [pltpu.SemaphoreType.DMA] * (3 * NBUF),
)
def _encode(table_hbm, x_hbm, aux_hbm, out_hbm, idx_v, rows_v, aux_v, *sems):
    sem_g = sems[0:NBUF]
    sem_a = sems[NBUF:2 * NBUF]
    sem_out = sems[2 * NBUF:3 * NBUF]
    wid = lax.axis_index("s") * NUM_CORES + lax.axis_index("c")
    base = wid * ROWS_PER_W
    pltpu.sync_copy(x_hbm.at[pl.ds(base, ROWS_PER_W)], idx_v)

    def fetch_start(k):
        b = k % NBUF
        g = pltpu.async_copy(
            table_hbm.at[idx_v.at[pl.ds(k * CHUNK, CHUNK)]], rows_v.at[b],
            sem_g[b])
        a = pltpu.async_copy(
            aux_hbm.at[pl.ds(base + k * CHUNK, CHUNK)], aux_v.at[b], sem_a[b])
        return g, a

    pending = {k: fetch_start(k) for k in range(min(NBUF - 1, NCHUNK))}
    pending_out = {}
    for k in range(NCHUNK):
        b = k % NBUF
        if k + NBUF - 1 < NCHUNK:
            if k >= 1:
                pending_out.pop(k - 1).wait()
            pending[k + NBUF - 1] = fetch_start(k + NBUF - 1)
        g, a = pending.pop(k)
        g.wait()
        a.wait()

        @plsc.parallel_loop(0, CHUNK, unroll=1)
        def _row(r):
            c2 = aux_v[b, r, pl.ds(NSEED * LANES, LANES)]
            ring = []
            for j in range(NSEED):
                vj = aux_v[b, r, pl.ds(j * LANES, LANES)]
                s = pl.ds(j * LANES, LANES)
                rows_v[b, r, s] = rows_v[b, r, s] + vj
                ring.append(vj)
            half = NSEED // 2
            for j in range(NSEED, NBLK):
                vn = c2 * ring[-half] - ring[-NSEED]
                s = pl.ds(j * LANES, LANES)
                rows_v[b, r, s] = rows_v[b, r, s] + vn
                ring.append(vn)
                ring.pop(0)

        pending_out[k] = pltpu.async_copy(
            rows_v.at[b], out_hbm.at[pl.ds(base + k * CHUNK, CHUNK)], sem_out[b])
    for k in sorted(pending_out):
        pending_out[k].wait()


def kernel(x, table):
    aux = jnp.asarray(_AUX)
    return _encode(table, x, aux)
